# baseline (speedup 1.0000x reference)
import functools

import jax
import jax.numpy as jnp
from jax.experimental import pallas as pl
from jax.experimental.pallas import tpu as pltpu

GRID = 16.0
MARGIN = 1.0
NUM_NEG = 16
P = 128
Q = 256
BIGF = 3.0e38
BIGI = 2**30


def _chunk_ids(q):
    cid = jax.lax.broadcasted_iota(jnp.int32, (Q, 1), 0) + q * Q
    cx = (cid % 32).astype(jnp.float32) * GRID + GRID / 2.0
    cy = (cid // 32).astype(jnp.float32) * GRID + GRID / 2.0
    return cid, cx, cy


def _top4_axis(p):
    ci = jax.lax.broadcasted_iota(jnp.int32, (32, 1), 0)
    cf = ci.astype(jnp.float32) * GRID + GRID / 2.0
    dd = (cf - p) * (cf - p)
    out = []
    for _ in range(4):
        m = jnp.min(dd, axis=0, keepdims=True)
        idx = jnp.min(jnp.where(dd == m, ci, BIGI), axis=0, keepdims=True)
        dd = jnp.where(ci == idx, BIGF, dd)
        out.append(idx)
    return out


def _nearest4(x, y, nq):
    del nq
    cols = _top4_axis(x)
    rows = _top4_axis(y)
    pm = x * x + y * y
    cand_v, cand_i = [], []
    for ri in rows:
        cyf = ri.astype(jnp.float32) * GRID + GRID / 2.0
        for cj in cols:
            cxf = cj.astype(jnp.float32) * GRID + GRID / 2.0
            cm = cxf * cxf + cyf * cyf
            d2 = (pm + cm) - 2.0 * (cxf * x + cyf * y)
            cand_v.append(jnp.sqrt(jnp.maximum(d2, 1e-12)))
            cand_i.append(ri * 32 + cj)
    V = jnp.concatenate(cand_v, axis=0)
    I = jnp.concatenate(cand_i, axis=0)
    out = []
    for _ in range(4):
        m = jnp.min(V, axis=0, keepdims=True)
        idx = jnp.min(jnp.where(V == m, I, BIGI), axis=0, keepdims=True)
        V = jnp.where(I == idx, BIGF, V)
        out.append(idx)
    return out


def _loss_kernel(kp_ref, wkp_ref, desc_ref, d2r_ref, homo_ref,
                 out_ref, n2_scr, *, nq, nsteps):
    @pl.when(pl.program_id(1) == 0)
    def _build_n2():
        for q in range(nq):
            d = d2r_ref[0, :, q * Q:(q + 1) * Q]
            rn = jnp.sqrt(jnp.sum(d * d, axis=0, keepdims=True))
            n2_scr[:, q * Q:(q + 1) * Q] = d / (rn + 1e-8)

    kp = kp_ref[0]
    x = jnp.transpose(kp[:, 0:1], (1, 0))
    y = jnp.transpose(kp[:, 1:2], (1, 0))

    ids1 = _nearest4(x, y, nq)
    h = homo_ref[0]
    ids16 = []
    for idj in ids1:
        cx = (idj % 32).astype(jnp.float32) * GRID + GRID / 2.0
        cy = (idj // 32).astype(jnp.float32) * GRID + GRID / 2.0
        wz = h[2:3, 0:1] * cx + h[2:3, 1:2] * cy + h[2:3, 2:3] + 1e-8
        wx = (h[0:1, 0:1] * cx + h[0:1, 1:2] * cy + h[0:1, 2:3]) / wz
        wy = (h[1:2, 0:1] * cx + h[1:2, 1:2] * cy + h[1:2, 2:3]) / wz
        ids16.extend(_nearest4(wx, wy, nq))

    dpc = desc_ref[0]
    n1pc = dpc / (jnp.sqrt(jnp.sum(dpc * dpc, axis=1, keepdims=True)) + 1e-8)

    wkp = wkp_ref[0]
    sx = jnp.clip(jnp.transpose(wkp[:, 0:1], (1, 0)) / GRID - 0.5, 0.0, 31.0)
    sy = jnp.clip(jnp.transpose(wkp[:, 1:2], (1, 0)) / GRID - 0.5, 0.0, 31.0)
    x0 = jnp.clip(jnp.floor(sx), 0.0, 30.0)
    y0 = jnp.clip(jnp.floor(sy), 0.0, 30.0)
    fx = sx - x0
    fy = sy - y0
    m00 = y0.astype(jnp.int32) * 32 + x0.astype(jnp.int32)
    w00 = (1.0 - fx) * (1.0 - fy)
    w01 = fx * (1.0 - fy)
    w10 = (1.0 - fx) * fy
    w11 = fx * fy

    samp = jnp.zeros((P, d2r_ref.shape[1]), jnp.float32)
    l0, l1, l2, l3 = [], [], [], []
    for q in range(nq):
        cid, _, _ = _chunk_ids(q)
        n2q = n2_scr[:, q * Q:(q + 1) * Q]
        g = jax.lax.dot_general(n2q, n1pc, (((0,), (1,)), ((), ())),
                                preferred_element_type=jnp.float32)
        sim = 2.0 - 2.0 * g
        eqs = [cid == idj for idj in ids16]
        while len(eqs) > 1:
            eqs = [a | b for a, b in zip(eqs[::2], eqs[1::2])]
        sim = jnp.where(eqs[0], BIGF, sim)
        s0, s1 = sim[:Q // 4], sim[Q // 4:Q // 2]
        s2, s3 = sim[Q // 2:3 * Q // 4], sim[3 * Q // 4:]
        a0, a1 = jnp.minimum(s0, s1), jnp.maximum(s0, s1)
        b0, b1 = jnp.minimum(s2, s3), jnp.maximum(s2, s3)
        c0, c2 = jnp.minimum(a0, b0), jnp.maximum(a0, b0)
        c1, c3 = jnp.minimum(a1, b1), jnp.maximum(a1, b1)
        d1, d2 = jnp.minimum(c2, c1), jnp.maximum(c2, c1)
        l0.append(c0)
        l1.append(d1)
        l2.append(d2)
        l3.append(c3)

        s_q = (w00 * (cid == m00) + w01 * (cid == m00 + 1)
               + w10 * (cid == m00 + 32) + w11 * (cid == m00 + 33))
        samp = samp + jax.lax.dot_general(
            s_q.astype(jnp.float32), d2r_ref[0, :, q * Q:(q + 1) * Q],
            (((0,), (1,)), ((), ())),
            preferred_element_type=jnp.float32)

    ns = jnp.sqrt(jnp.sum(samp * samp, axis=1, keepdims=True))
    nsamp = samp / (ns + 1e-8)
    posc = 2.0 - 2.0 * jnp.sum(n1pc * nsamp, axis=1, keepdims=True)
    pos = jnp.transpose(posc, (1, 0))

    A = jnp.concatenate(l0, axis=0)
    S2 = jnp.concatenate(l1, axis=0)
    S3 = jnp.concatenate(l2, axis=0)
    S4 = jnp.concatenate(l3, axis=0)
    acc = jnp.zeros((1, P), jnp.float32)
    for _ in range(NUM_NEG):
        m = jnp.min(A, axis=0, keepdims=True)
        eq = A == m
        A = jnp.where(eq, S2, A)
        S2 = jnp.where(eq, S3, S2)
        S3 = jnp.where(eq, S4, S3)
        S4 = jnp.where(eq, BIGF, S4)
        acc = acc + jnp.maximum(pos - m + MARGIN, 0.0)

    step = pl.program_id(0) * pl.num_programs(1) + pl.program_id(1)

    @pl.when(step == 0)
    def _init_out():
        out_ref[:, :] = jnp.zeros((1, 1), jnp.float32)

    part = jnp.sum(acc, axis=1, keepdims=True)
    upd = out_ref[:, :] + part

    @pl.when(step == nsteps - 1)
    def _scale_out():
        out_ref[:, :] = upd / (nsteps * P * NUM_NEG)

    @pl.when(step < nsteps - 1)
    def _acc_out():
        out_ref[:, :] = upd


@jax.jit
def kernel(kp1, w_kp1, kp1_desc, desc2, homo12):
    b, n, c = kp1_desc.shape
    _, _, hh, ww = desc2.shape
    m = hh * ww
    nq = m // Q
    nb = n // P

    d2r = desc2.reshape(b, c, m)

    grid = (b, nb)
    out = pl.pallas_call(
        functools.partial(_loss_kernel, nq=nq, nsteps=b * nb),
        grid=grid,
        in_specs=[
            pl.BlockSpec((1, P, 2), lambda bi, ri: (bi, ri, 0)),
            pl.BlockSpec((1, P, 2), lambda bi, ri: (bi, ri, 0)),
            pl.BlockSpec((1, P, c), lambda bi, ri: (bi, ri, 0)),
            pl.BlockSpec((1, c, m), lambda bi, ri: (bi, 0, 0)),
            pl.BlockSpec((1, 3, 3), lambda bi, ri: (bi, 0, 0)),
        ],
        out_specs=pl.BlockSpec((1, 1), lambda bi, ri: (0, 0)),
        out_shape=jax.ShapeDtypeStruct((1, 1), jnp.float32),
        scratch_shapes=[pltpu.VMEM((c, m), jnp.float32)],
    )(kp1, w_kp1, kp1_desc, d2r, homo12)

    return out.reshape(())


# speedup vs baseline: 1.1586x; 1.1586x over previous
import functools

import jax
import jax.numpy as jnp
from jax.experimental import pallas as pl
from jax.experimental.pallas import tpu as pltpu

GRID = 16.0
MARGIN = 1.0
NUM_NEG = 16
P = 128
Q = 256
BIGF = 3.0e38
BIGI = 2**30


def _chunk_ids(q):
    cid = jax.lax.broadcasted_iota(jnp.int32, (Q, 1), 0) + q * Q
    cx = (cid % 32).astype(jnp.float32) * GRID + GRID / 2.0
    cy = (cid // 32).astype(jnp.float32) * GRID + GRID / 2.0
    return cid, cx, cy


def _top4_axis(p):
    ci = jax.lax.broadcasted_iota(jnp.int32, (32, 1), 0)
    cf = ci.astype(jnp.float32) * GRID + GRID / 2.0
    dd = (cf - p) * (cf - p)
    out = []
    for _ in range(4):
        m = jnp.min(dd, axis=0, keepdims=True)
        idx = jnp.min(jnp.where(dd == m, ci, BIGI), axis=0, keepdims=True)
        dd = jnp.where(ci == idx, BIGF, dd)
        out.append(idx)
    return out


def _nearest4(x, y, nq):
    del nq
    cols = _top4_axis(x)
    rows = _top4_axis(y)
    pm = x * x + y * y
    cand_v, cand_i = [], []
    for ri in rows:
        cyf = ri.astype(jnp.float32) * GRID + GRID / 2.0
        for cj in cols:
            cxf = cj.astype(jnp.float32) * GRID + GRID / 2.0
            cm = cxf * cxf + cyf * cyf
            d2 = (pm + cm) - 2.0 * (cxf * x + cyf * y)
            cand_v.append(jnp.sqrt(jnp.maximum(d2, 1e-12)))
            cand_i.append(ri * 32 + cj)
    V = jnp.concatenate(cand_v, axis=0)
    I = jnp.concatenate(cand_i, axis=0)
    out = []
    for _ in range(4):
        m = jnp.min(V, axis=0, keepdims=True)
        idx = jnp.min(jnp.where(V == m, I, BIGI), axis=0, keepdims=True)
        V = jnp.where(I == idx, BIGF, V)
        out.append(idx)
    return out


def _loss_kernel(kp_ref, wkp_ref, desc_ref, d2r_ref, homo_ref,
                 out_ref, n2_scr, *, nq, nsteps):
    @pl.when(pl.program_id(1) == 0)
    def _build_n2():
        for q in range(nq):
            d = d2r_ref[0, :, q * Q:(q + 1) * Q]
            rn = jnp.sqrt(jnp.sum(d * d, axis=0, keepdims=True))
            n2_scr[:, q * Q:(q + 1) * Q] = d / (rn + 1e-8)

    x = kp_ref[0, 0:1, :]
    y = kp_ref[0, 1:2, :]

    ids1 = _nearest4(x, y, nq)
    h = homo_ref[0]
    ids16 = []
    for idj in ids1:
        cx = (idj % 32).astype(jnp.float32) * GRID + GRID / 2.0
        cy = (idj // 32).astype(jnp.float32) * GRID + GRID / 2.0
        wz = h[2:3, 0:1] * cx + h[2:3, 1:2] * cy + h[2:3, 2:3] + 1e-8
        wx = (h[0:1, 0:1] * cx + h[0:1, 1:2] * cy + h[0:1, 2:3]) / wz
        wy = (h[1:2, 0:1] * cx + h[1:2, 1:2] * cy + h[1:2, 2:3]) / wz
        ids16.extend(_nearest4(wx, wy, nq))

    dpc = desc_ref[0]
    n1pc = dpc / (jnp.sqrt(jnp.sum(dpc * dpc, axis=1, keepdims=True)) + 1e-8)

    sx = jnp.clip(wkp_ref[0, 0:1, :] / GRID - 0.5, 0.0, 31.0)
    sy = jnp.clip(wkp_ref[0, 1:2, :] / GRID - 0.5, 0.0, 31.0)
    x0 = jnp.clip(jnp.floor(sx), 0.0, 30.0)
    y0 = jnp.clip(jnp.floor(sy), 0.0, 30.0)
    fx = sx - x0
    fy = sy - y0
    m00 = y0.astype(jnp.int32) * 32 + x0.astype(jnp.int32)
    w00 = (1.0 - fx) * (1.0 - fy)
    w01 = fx * (1.0 - fy)
    w10 = (1.0 - fx) * fy
    w11 = fx * fy

    samp = jnp.zeros((P, d2r_ref.shape[1]), jnp.float32)
    l0, l1, l2, l3 = [], [], [], []
    for q in range(nq):
        cid, _, _ = _chunk_ids(q)
        n2q = n2_scr[:, q * Q:(q + 1) * Q]
        g = jax.lax.dot_general(n2q, n1pc, (((0,), (1,)), ((), ())),
                                preferred_element_type=jnp.float32)
        sim = 2.0 - 2.0 * g
        eqs = [cid == idj for idj in ids16]
        while len(eqs) > 1:
            eqs = [a | b for a, b in zip(eqs[::2], eqs[1::2])]
        sim = jnp.where(eqs[0], BIGF, sim)
        s0, s1 = sim[:Q // 4], sim[Q // 4:Q // 2]
        s2, s3 = sim[Q // 2:3 * Q // 4], sim[3 * Q // 4:]
        a0, a1 = jnp.minimum(s0, s1), jnp.maximum(s0, s1)
        b0, b1 = jnp.minimum(s2, s3), jnp.maximum(s2, s3)
        c0, c2 = jnp.minimum(a0, b0), jnp.maximum(a0, b0)
        c1, c3 = jnp.minimum(a1, b1), jnp.maximum(a1, b1)
        d1, d2 = jnp.minimum(c2, c1), jnp.maximum(c2, c1)
        l0.append(c0)
        l1.append(d1)
        l2.append(d2)
        l3.append(c3)

        s_q = (w00 * (cid == m00) + w01 * (cid == m00 + 1)
               + w10 * (cid == m00 + 32) + w11 * (cid == m00 + 33))
        samp = samp + jax.lax.dot_general(
            s_q.astype(jnp.float32), d2r_ref[0, :, q * Q:(q + 1) * Q],
            (((0,), (1,)), ((), ())),
            preferred_element_type=jnp.float32)

    ns = jnp.sqrt(jnp.sum(samp * samp, axis=1, keepdims=True))
    nsamp = samp / (ns + 1e-8)
    posc = 2.0 - 2.0 * jnp.sum(n1pc * nsamp, axis=1, keepdims=True)
    pos = jnp.transpose(posc, (1, 0))

    A = jnp.concatenate(l0, axis=0)
    S2 = jnp.concatenate(l1, axis=0)
    S3 = jnp.concatenate(l2, axis=0)
    S4 = jnp.concatenate(l3, axis=0)
    acc = jnp.zeros((1, P), jnp.float32)
    for _ in range(NUM_NEG):
        m = jnp.min(A, axis=0, keepdims=True)
        eq = A == m
        A = jnp.where(eq, S2, A)
        S2 = jnp.where(eq, S3, S2)
        S3 = jnp.where(eq, S4, S3)
        S4 = jnp.where(eq, BIGF, S4)
        acc = acc + jnp.maximum(pos - m + MARGIN, 0.0)

    step = pl.program_id(0) * pl.num_programs(1) + pl.program_id(1)

    @pl.when(step == 0)
    def _init_out():
        out_ref[:, :] = jnp.zeros((1, 1), jnp.float32)

    part = jnp.sum(acc, axis=1, keepdims=True)
    upd = out_ref[:, :] + part

    @pl.when(step == nsteps - 1)
    def _scale_out():
        out_ref[:, :] = upd / (nsteps * P * NUM_NEG)

    @pl.when(step < nsteps - 1)
    def _acc_out():
        out_ref[:, :] = upd


@jax.jit
def kernel(kp1, w_kp1, kp1_desc, desc2, homo12):
    b, n, c = kp1_desc.shape
    _, _, hh, ww = desc2.shape
    m = hh * ww
    nq = m // Q
    nb = n // P

    kpt = jnp.transpose(kp1, (0, 2, 1))
    wkpt = jnp.transpose(w_kp1, (0, 2, 1))
    d2r = desc2.reshape(b, c, m)

    grid = (b, nb)
    out = pl.pallas_call(
        functools.partial(_loss_kernel, nq=nq, nsteps=b * nb),
        grid=grid,
        in_specs=[
            pl.BlockSpec((1, 2, P), lambda bi, ri: (bi, 0, ri)),
            pl.BlockSpec((1, 2, P), lambda bi, ri: (bi, 0, ri)),
            pl.BlockSpec((1, P, c), lambda bi, ri: (bi, ri, 0)),
            pl.BlockSpec((1, c, m), lambda bi, ri: (bi, 0, 0)),
            pl.BlockSpec((1, 3, 3), lambda bi, ri: (bi, 0, 0)),
        ],
        out_specs=pl.BlockSpec((1, 1), lambda bi, ri: (0, 0)),
        out_shape=jax.ShapeDtypeStruct((1, 1), jnp.float32),
        scratch_shapes=[pltpu.VMEM((c, m), jnp.float32)],
    )(kpt, wkpt, kp1_desc, d2r, homo12)

    return out.reshape(())


# speedup vs baseline: 1.2705x; 1.0965x over previous
import functools

import jax
import jax.numpy as jnp
from jax.experimental import pallas as pl
from jax.experimental.pallas import tpu as pltpu

GRID = 16.0
MARGIN = 1.0
NUM_NEG = 16
P = 256
Q = 256
BIGF = 3.0e38
BIGI = 2**30


def _chunk_ids(q):
    cid = jax.lax.broadcasted_iota(jnp.int32, (Q, 1), 0) + q * Q
    cx = (cid % 32).astype(jnp.float32) * GRID + GRID / 2.0
    cy = (cid // 32).astype(jnp.float32) * GRID + GRID / 2.0
    return cid, cx, cy


def _top4_axis(p):
    ci = jax.lax.broadcasted_iota(jnp.int32, (32, 1), 0)
    cf = ci.astype(jnp.float32) * GRID + GRID / 2.0
    dd = (cf - p) * (cf - p)
    out = []
    for _ in range(4):
        m = jnp.min(dd, axis=0, keepdims=True)
        idx = jnp.min(jnp.where(dd == m, ci, BIGI), axis=0, keepdims=True)
        dd = jnp.where(ci == idx, BIGF, dd)
        out.append(idx)
    return out


def _nearest4(x, y, nq):
    del nq
    cols = _top4_axis(x)
    rows = _top4_axis(y)
    pm = x * x + y * y
    cand_v, cand_i = [], []
    for ri in rows:
        cyf = ri.astype(jnp.float32) * GRID + GRID / 2.0
        for cj in cols:
            cxf = cj.astype(jnp.float32) * GRID + GRID / 2.0
            cm = cxf * cxf + cyf * cyf
            d2 = (pm + cm) - 2.0 * (cxf * x + cyf * y)
            cand_v.append(jnp.sqrt(jnp.maximum(d2, 1e-12)))
            cand_i.append(ri * 32 + cj)
    V = jnp.concatenate(cand_v, axis=0)
    I = jnp.concatenate(cand_i, axis=0)
    out = []
    for _ in range(4):
        m = jnp.min(V, axis=0, keepdims=True)
        idx = jnp.min(jnp.where(V == m, I, BIGI), axis=0, keepdims=True)
        V = jnp.where(I == idx, BIGF, V)
        out.append(idx)
    return out


def _loss_kernel(kp_ref, wkp_ref, desc_ref, d2r_ref, homo_ref,
                 out_ref, n2_scr, *, nq, nsteps):
    @pl.when(pl.program_id(1) == 0)
    def _build_n2():
        for q in range(nq):
            d = d2r_ref[0, :, q * Q:(q + 1) * Q]
            rn = jnp.sqrt(jnp.sum(d * d, axis=0, keepdims=True))
            n2_scr[:, q * Q:(q + 1) * Q] = d / (rn + 1e-8)

    x = kp_ref[0, 0:1, :]
    y = kp_ref[0, 1:2, :]

    ids1 = _nearest4(x, y, nq)
    h = homo_ref[0]
    ids16 = []
    for idj in ids1:
        cx = (idj % 32).astype(jnp.float32) * GRID + GRID / 2.0
        cy = (idj // 32).astype(jnp.float32) * GRID + GRID / 2.0
        wz = h[2:3, 0:1] * cx + h[2:3, 1:2] * cy + h[2:3, 2:3] + 1e-8
        wx = (h[0:1, 0:1] * cx + h[0:1, 1:2] * cy + h[0:1, 2:3]) / wz
        wy = (h[1:2, 0:1] * cx + h[1:2, 1:2] * cy + h[1:2, 2:3]) / wz
        ids16.extend(_nearest4(wx, wy, nq))

    dpc = desc_ref[0]
    n1pc = dpc / (jnp.sqrt(jnp.sum(dpc * dpc, axis=1, keepdims=True)) + 1e-8)

    sx = jnp.clip(wkp_ref[0, 0:1, :] / GRID - 0.5, 0.0, 31.0)
    sy = jnp.clip(wkp_ref[0, 1:2, :] / GRID - 0.5, 0.0, 31.0)
    x0 = jnp.clip(jnp.floor(sx), 0.0, 30.0)
    y0 = jnp.clip(jnp.floor(sy), 0.0, 30.0)
    fx = sx - x0
    fy = sy - y0
    m00 = y0.astype(jnp.int32) * 32 + x0.astype(jnp.int32)
    w00 = (1.0 - fx) * (1.0 - fy)
    w01 = fx * (1.0 - fy)
    w10 = (1.0 - fx) * fy
    w11 = fx * fy

    samp = jnp.zeros((P, d2r_ref.shape[1]), jnp.float32)
    l0, l1, l2, l3 = [], [], [], []
    for q in range(nq):
        cid, _, _ = _chunk_ids(q)
        n2q = n2_scr[:, q * Q:(q + 1) * Q]
        g = jax.lax.dot_general(n2q, n1pc, (((0,), (1,)), ((), ())),
                                preferred_element_type=jnp.float32)
        sim = 2.0 - 2.0 * g
        eqs = [cid == idj for idj in ids16]
        while len(eqs) > 1:
            eqs = [a | b for a, b in zip(eqs[::2], eqs[1::2])]
        sim = jnp.where(eqs[0], BIGF, sim)
        s0, s1 = sim[:Q // 4], sim[Q // 4:Q // 2]
        s2, s3 = sim[Q // 2:3 * Q // 4], sim[3 * Q // 4:]
        a0, a1 = jnp.minimum(s0, s1), jnp.maximum(s0, s1)
        b0, b1 = jnp.minimum(s2, s3), jnp.maximum(s2, s3)
        c0, c2 = jnp.minimum(a0, b0), jnp.maximum(a0, b0)
        c1, c3 = jnp.minimum(a1, b1), jnp.maximum(a1, b1)
        d1, d2 = jnp.minimum(c2, c1), jnp.maximum(c2, c1)
        l0.append(c0)
        l1.append(d1)
        l2.append(d2)
        l3.append(c3)

        s_q = (w00 * (cid == m00) + w01 * (cid == m00 + 1)
               + w10 * (cid == m00 + 32) + w11 * (cid == m00 + 33))
        samp = samp + jax.lax.dot_general(
            s_q.astype(jnp.float32), d2r_ref[0, :, q * Q:(q + 1) * Q],
            (((0,), (1,)), ((), ())),
            preferred_element_type=jnp.float32)

    ns = jnp.sqrt(jnp.sum(samp * samp, axis=1, keepdims=True))
    nsamp = samp / (ns + 1e-8)
    posc = 2.0 - 2.0 * jnp.sum(n1pc * nsamp, axis=1, keepdims=True)
    pos = jnp.transpose(posc, (1, 0))

    A = jnp.concatenate(l0, axis=0)
    S2 = jnp.concatenate(l1, axis=0)
    S3 = jnp.concatenate(l2, axis=0)
    S4 = jnp.concatenate(l3, axis=0)
    acc = jnp.zeros((1, P), jnp.float32)
    for _ in range(NUM_NEG):
        m = jnp.min(A, axis=0, keepdims=True)
        eq = A == m
        A = jnp.where(eq, S2, A)
        S2 = jnp.where(eq, S3, S2)
        S3 = jnp.where(eq, S4, S3)
        S4 = jnp.where(eq, BIGF, S4)
        acc = acc + jnp.maximum(pos - m + MARGIN, 0.0)

    step = pl.program_id(0) * pl.num_programs(1) + pl.program_id(1)

    @pl.when(step == 0)
    def _init_out():
        out_ref[:, :] = jnp.zeros((1, 1), jnp.float32)

    part = jnp.sum(acc, axis=1, keepdims=True)
    upd = out_ref[:, :] + part

    @pl.when(step == nsteps - 1)
    def _scale_out():
        out_ref[:, :] = upd / (nsteps * P * NUM_NEG)

    @pl.when(step < nsteps - 1)
    def _acc_out():
        out_ref[:, :] = upd


@jax.jit
def kernel(kp1, w_kp1, kp1_desc, desc2, homo12):
    b, n, c = kp1_desc.shape
    _, _, hh, ww = desc2.shape
    m = hh * ww
    nq = m // Q
    nb = n // P

    kpt = jnp.transpose(kp1, (0, 2, 1))
    wkpt = jnp.transpose(w_kp1, (0, 2, 1))
    d2r = desc2.reshape(b, c, m)

    grid = (b, nb)
    out = pl.pallas_call(
        functools.partial(_loss_kernel, nq=nq, nsteps=b * nb),
        grid=grid,
        in_specs=[
            pl.BlockSpec((1, 2, P), lambda bi, ri: (bi, 0, ri)),
            pl.BlockSpec((1, 2, P), lambda bi, ri: (bi, 0, ri)),
            pl.BlockSpec((1, P, c), lambda bi, ri: (bi, ri, 0)),
            pl.BlockSpec((1, c, m), lambda bi, ri: (bi, 0, 0)),
            pl.BlockSpec((1, 3, 3), lambda bi, ri: (bi, 0, 0)),
        ],
        out_specs=pl.BlockSpec((1, 1), lambda bi, ri: (0, 0)),
        out_shape=jax.ShapeDtypeStruct((1, 1), jnp.float32),
        scratch_shapes=[pltpu.VMEM((c, m), jnp.float32)],
    )(kpt, wkpt, kp1_desc, d2r, homo12)

    return out.reshape(())


# speedup vs baseline: 1.2831x; 1.0100x over previous
import functools

import jax
import jax.numpy as jnp
from jax.experimental import pallas as pl
from jax.experimental.pallas import tpu as pltpu

GRID = 16.0
MARGIN = 1.0
NUM_NEG = 16
P = 256
Q = 256
BIGF = 3.0e38
BIGI = 2**30


def _chunk_ids(q):
    cid = jax.lax.broadcasted_iota(jnp.int32, (Q, 1), 0) + q * Q
    cx = (cid % 32).astype(jnp.float32) * GRID + GRID / 2.0
    cy = (cid // 32).astype(jnp.float32) * GRID + GRID / 2.0
    return cid, cx, cy


def _top4_axis(p):
    ci = jax.lax.broadcasted_iota(jnp.int32, (32, 1), 0)
    cf = ci.astype(jnp.float32) * GRID + GRID / 2.0
    dd = (cf - p) * (cf - p)
    out = []
    for _ in range(4):
        m = jnp.min(dd, axis=0, keepdims=True)
        idx = jnp.min(jnp.where(dd == m, ci, BIGI), axis=0, keepdims=True)
        dd = jnp.where(ci == idx, BIGF, dd)
        out.append(idx)
    return out


def _nearest4(x, y, nq):
    del nq
    cols = _top4_axis(x)
    rows = _top4_axis(y)
    pm = x * x + y * y
    cand_v, cand_i = [], []
    for ri in rows:
        cyf = ri.astype(jnp.float32) * GRID + GRID / 2.0
        for cj in cols:
            cxf = cj.astype(jnp.float32) * GRID + GRID / 2.0
            cm = cxf * cxf + cyf * cyf
            d2 = (pm + cm) - 2.0 * (cxf * x + cyf * y)
            cand_v.append(jnp.sqrt(jnp.maximum(d2, 1e-12)))
            cand_i.append(ri * 32 + cj)
    V = jnp.concatenate(cand_v, axis=0)
    I = jnp.concatenate(cand_i, axis=0)
    out = []
    for _ in range(4):
        m = jnp.min(V, axis=0, keepdims=True)
        idx = jnp.min(jnp.where(V == m, I, BIGI), axis=0, keepdims=True)
        V = jnp.where(I == idx, BIGF, V)
        out.append(idx)
    return out


def _loss_kernel(kp_ref, wkp_ref, desc_ref, d2r_ref, homo_ref,
                 out_ref, n2_scr, *, nq, nsteps):
    @pl.when(pl.program_id(1) == 0)
    def _build_n2():
        for q in range(nq):
            d = d2r_ref[0, :, q * Q:(q + 1) * Q]
            rn = jnp.sqrt(jnp.sum(d * d, axis=0, keepdims=True))
            n2_scr[:, q * Q:(q + 1) * Q] = d / (rn + 1e-8)

    x = kp_ref[0, 0:1, :]
    y = kp_ref[0, 1:2, :]

    ids1 = _nearest4(x, y, nq)
    h = homo_ref[0]
    ids16 = []
    for idj in ids1:
        cx = (idj % 32).astype(jnp.float32) * GRID + GRID / 2.0
        cy = (idj // 32).astype(jnp.float32) * GRID + GRID / 2.0
        wz = h[2:3, 0:1] * cx + h[2:3, 1:2] * cy + h[2:3, 2:3] + 1e-8
        wx = (h[0:1, 0:1] * cx + h[0:1, 1:2] * cy + h[0:1, 2:3]) / wz
        wy = (h[1:2, 0:1] * cx + h[1:2, 1:2] * cy + h[1:2, 2:3]) / wz
        ids16.extend(_nearest4(wx, wy, nq))

    dpc = desc_ref[0]
    n1pc = dpc / (jnp.sqrt(jnp.sum(dpc * dpc, axis=1, keepdims=True)) + 1e-8)

    sx = jnp.clip(wkp_ref[0, 0:1, :] / GRID - 0.5, 0.0, 31.0)
    sy = jnp.clip(wkp_ref[0, 1:2, :] / GRID - 0.5, 0.0, 31.0)
    x0 = jnp.clip(jnp.floor(sx), 0.0, 30.0)
    y0 = jnp.clip(jnp.floor(sy), 0.0, 30.0)
    fx = sx - x0
    fy = sy - y0
    m00 = y0.astype(jnp.int32) * 32 + x0.astype(jnp.int32)
    w00 = (1.0 - fx) * (1.0 - fy)
    w01 = fx * (1.0 - fy)
    w10 = (1.0 - fx) * fy
    w11 = fx * fy

    half_cands = []
    nh = max(1, nq // 2)
    for hh in range(0, nq, nh):
        l0, l1, l2, l3 = [], [], [], []
        for q in range(hh, hh + nh):
            cid, _, _ = _chunk_ids(q)
            n2q = n2_scr[:, q * Q:(q + 1) * Q]
            g = jax.lax.dot_general(n2q, n1pc, (((0,), (1,)), ((), ())),
                                    preferred_element_type=jnp.float32)
            sim = 2.0 - 2.0 * g
            eqs = [cid == idj for idj in ids16]
            while len(eqs) > 1:
                eqs = [a | b for a, b in zip(eqs[::2], eqs[1::2])]
            sim = jnp.where(eqs[0], BIGF, sim)
            s0, s1 = sim[:Q // 4], sim[Q // 4:Q // 2]
            s2, s3 = sim[Q // 2:3 * Q // 4], sim[3 * Q // 4:]
            a0, a1 = jnp.minimum(s0, s1), jnp.maximum(s0, s1)
            b0, b1 = jnp.minimum(s2, s3), jnp.maximum(s2, s3)
            c0, c2 = jnp.minimum(a0, b0), jnp.maximum(a0, b0)
            c1, c3 = jnp.minimum(a1, b1), jnp.maximum(a1, b1)
            d1, d2 = jnp.minimum(c2, c1), jnp.maximum(c2, c1)
            l0.append(c0)
            l1.append(d1)
            l2.append(d2)
            l3.append(c3)

        A = jnp.concatenate(l0, axis=0)
        S2 = jnp.concatenate(l1, axis=0)
        S3 = jnp.concatenate(l2, axis=0)
        S4 = jnp.concatenate(l3, axis=0)
        for _ in range(NUM_NEG):
            m = jnp.min(A, axis=0, keepdims=True)
            eq = A == m
            A = jnp.where(eq, S2, A)
            S2 = jnp.where(eq, S3, S2)
            S3 = jnp.where(eq, S4, S3)
            S4 = jnp.where(eq, BIGF, S4)
            half_cands.append(m)

    k = len(half_cands) // 2
    U = jnp.concatenate(half_cands[:k], axis=0)
    W = jnp.concatenate(half_cands[k:], axis=0)
    A = jnp.minimum(U, W)
    S2 = jnp.maximum(U, W)
    negs = []
    for _ in range(NUM_NEG):
        m = jnp.min(A, axis=0, keepdims=True)
        eq = A == m
        A = jnp.where(eq, S2, A)
        S2 = jnp.where(eq, BIGF, S2)
        negs.append(m)

    samp = jnp.zeros((P, d2r_ref.shape[1]), jnp.float32)
    for q in range(nq):
        cid, _, _ = _chunk_ids(q)
        s_q = (w00 * (cid == m00) + w01 * (cid == m00 + 1)
               + w10 * (cid == m00 + 32) + w11 * (cid == m00 + 33))
        samp = samp + jax.lax.dot_general(
            s_q.astype(jnp.float32), d2r_ref[0, :, q * Q:(q + 1) * Q],
            (((0,), (1,)), ((), ())),
            preferred_element_type=jnp.float32)

    ns = jnp.sqrt(jnp.sum(samp * samp, axis=1, keepdims=True))
    nsamp = samp / (ns + 1e-8)
    posc = 2.0 - 2.0 * jnp.sum(n1pc * nsamp, axis=1, keepdims=True)
    pos = jnp.transpose(posc, (1, 0))

    acc = jnp.zeros((1, P), jnp.float32)
    for m in negs:
        acc = acc + jnp.maximum(pos - m + MARGIN, 0.0)

    step = pl.program_id(0) * pl.num_programs(1) + pl.program_id(1)

    @pl.when(step == 0)
    def _init_out():
        out_ref[:, :] = jnp.zeros((1, 1), jnp.float32)

    part = jnp.sum(acc, axis=1, keepdims=True)
    upd = out_ref[:, :] + part

    @pl.when(step == nsteps - 1)
    def _scale_out():
        out_ref[:, :] = upd / (nsteps * P * NUM_NEG)

    @pl.when(step < nsteps - 1)
    def _acc_out():
        out_ref[:, :] = upd


@jax.jit
def kernel(kp1, w_kp1, kp1_desc, desc2, homo12):
    b, n, c = kp1_desc.shape
    _, _, hh, ww = desc2.shape
    m = hh * ww
    nq = m // Q
    nb = n // P

    kpt = jnp.transpose(kp1, (0, 2, 1))
    wkpt = jnp.transpose(w_kp1, (0, 2, 1))
    d2r = desc2.reshape(b, c, m)

    grid = (b, nb)
    out = pl.pallas_call(
        functools.partial(_loss_kernel, nq=nq, nsteps=b * nb),
        grid=grid,
        in_specs=[
            pl.BlockSpec((1, 2, P), lambda bi, ri: (bi, 0, ri)),
            pl.BlockSpec((1, 2, P), lambda bi, ri: (bi, 0, ri)),
            pl.BlockSpec((1, P, c), lambda bi, ri: (bi, ri, 0)),
            pl.BlockSpec((1, c, m), lambda bi, ri: (bi, 0, 0)),
            pl.BlockSpec((1, 3, 3), lambda bi, ri: (bi, 0, 0)),
        ],
        out_specs=pl.BlockSpec((1, 1), lambda bi, ri: (0, 0)),
        out_shape=jax.ShapeDtypeStruct((1, 1), jnp.float32),
        scratch_shapes=[pltpu.VMEM((c, m), jnp.float32)],
    )(kpt, wkpt, kp1_desc, d2r, homo12)

    return out.reshape(())


# speedup vs baseline: 1.2844x; 1.0010x over previous
import functools

import jax
import jax.numpy as jnp
from jax.experimental import pallas as pl
from jax.experimental.pallas import tpu as pltpu

GRID = 16.0
MARGIN = 1.0
NUM_NEG = 16
P = 256
Q = 512
BIGF = 3.0e38
BIGI = 2**30


def _chunk_ids(q):
    cid = jax.lax.broadcasted_iota(jnp.int32, (Q, 1), 0) + q * Q
    cx = (cid % 32).astype(jnp.float32) * GRID + GRID / 2.0
    cy = (cid // 32).astype(jnp.float32) * GRID + GRID / 2.0
    return cid, cx, cy


def _top4_axis(p):
    ci = jax.lax.broadcasted_iota(jnp.int32, (32, 1), 0)
    cf = ci.astype(jnp.float32) * GRID + GRID / 2.0
    dd = (cf - p) * (cf - p)
    out = []
    for _ in range(4):
        m = jnp.min(dd, axis=0, keepdims=True)
        idx = jnp.min(jnp.where(dd == m, ci, BIGI), axis=0, keepdims=True)
        dd = jnp.where(ci == idx, BIGF, dd)
        out.append(idx)
    return out


def _nearest4(x, y, nq):
    del nq
    cols = _top4_axis(x)
    rows = _top4_axis(y)
    pm = x * x + y * y
    cand_v, cand_i = [], []
    for ri in rows:
        cyf = ri.astype(jnp.float32) * GRID + GRID / 2.0
        for cj in cols:
            cxf = cj.astype(jnp.float32) * GRID + GRID / 2.0
            cm = cxf * cxf + cyf * cyf
            d2 = (pm + cm) - 2.0 * (cxf * x + cyf * y)
            cand_v.append(jnp.sqrt(jnp.maximum(d2, 1e-12)))
            cand_i.append(ri * 32 + cj)
    V = jnp.concatenate(cand_v, axis=0)
    I = jnp.concatenate(cand_i, axis=0)
    out = []
    for _ in range(4):
        m = jnp.min(V, axis=0, keepdims=True)
        idx = jnp.min(jnp.where(V == m, I, BIGI), axis=0, keepdims=True)
        V = jnp.where(I == idx, BIGF, V)
        out.append(idx)
    return out


def _loss_kernel(kp_ref, wkp_ref, desc_ref, d2r_ref, homo_ref,
                 out_ref, n2_scr, *, nq, nsteps):
    @pl.when(pl.program_id(1) == 0)
    def _build_n2():
        for q in range(nq):
            d = d2r_ref[0, :, q * Q:(q + 1) * Q]
            rn = jnp.sqrt(jnp.sum(d * d, axis=0, keepdims=True))
            n2_scr[:, q * Q:(q + 1) * Q] = d / (rn + 1e-8)

    x = kp_ref[0, 0:1, :]
    y = kp_ref[0, 1:2, :]

    ids1 = _nearest4(x, y, nq)
    h = homo_ref[0]
    ids16 = []
    for idj in ids1:
        cx = (idj % 32).astype(jnp.float32) * GRID + GRID / 2.0
        cy = (idj // 32).astype(jnp.float32) * GRID + GRID / 2.0
        wz = h[2:3, 0:1] * cx + h[2:3, 1:2] * cy + h[2:3, 2:3] + 1e-8
        wx = (h[0:1, 0:1] * cx + h[0:1, 1:2] * cy + h[0:1, 2:3]) / wz
        wy = (h[1:2, 0:1] * cx + h[1:2, 1:2] * cy + h[1:2, 2:3]) / wz
        ids16.extend(_nearest4(wx, wy, nq))

    dpc = desc_ref[0]
    n1pc = dpc / (jnp.sqrt(jnp.sum(dpc * dpc, axis=1, keepdims=True)) + 1e-8)

    sx = jnp.clip(wkp_ref[0, 0:1, :] / GRID - 0.5, 0.0, 31.0)
    sy = jnp.clip(wkp_ref[0, 1:2, :] / GRID - 0.5, 0.0, 31.0)
    x0 = jnp.clip(jnp.floor(sx), 0.0, 30.0)
    y0 = jnp.clip(jnp.floor(sy), 0.0, 30.0)
    fx = sx - x0
    fy = sy - y0
    m00 = y0.astype(jnp.int32) * 32 + x0.astype(jnp.int32)
    w00 = (1.0 - fx) * (1.0 - fy)
    w01 = fx * (1.0 - fy)
    w10 = (1.0 - fx) * fy
    w11 = fx * fy

    half_cands = []
    nh = max(1, nq // 2)
    for hh in range(0, nq, nh):
        l0, l1, l2, l3 = [], [], [], []
        for q in range(hh, hh + nh):
            cid, _, _ = _chunk_ids(q)
            n2q = n2_scr[:, q * Q:(q + 1) * Q]
            g = jax.lax.dot_general(n2q, n1pc, (((0,), (1,)), ((), ())),
                                    preferred_element_type=jnp.float32)
            sim = 2.0 - 2.0 * g
            eqs = [cid == idj for idj in ids16]
            while len(eqs) > 1:
                eqs = [a | b for a, b in zip(eqs[::2], eqs[1::2])]
            sim = jnp.where(eqs[0], BIGF, sim)
            s0, s1 = sim[:Q // 4], sim[Q // 4:Q // 2]
            s2, s3 = sim[Q // 2:3 * Q // 4], sim[3 * Q // 4:]
            a0, a1 = jnp.minimum(s0, s1), jnp.maximum(s0, s1)
            b0, b1 = jnp.minimum(s2, s3), jnp.maximum(s2, s3)
            c0, c2 = jnp.minimum(a0, b0), jnp.maximum(a0, b0)
            c1, c3 = jnp.minimum(a1, b1), jnp.maximum(a1, b1)
            d1, d2 = jnp.minimum(c2, c1), jnp.maximum(c2, c1)
            l0.append(c0)
            l1.append(d1)
            l2.append(d2)
            l3.append(c3)

        A = jnp.concatenate(l0, axis=0)
        S2 = jnp.concatenate(l1, axis=0)
        S3 = jnp.concatenate(l2, axis=0)
        S4 = jnp.concatenate(l3, axis=0)
        for _ in range(NUM_NEG):
            m = jnp.min(A, axis=0, keepdims=True)
            eq = A == m
            A = jnp.where(eq, S2, A)
            S2 = jnp.where(eq, S3, S2)
            S3 = jnp.where(eq, S4, S3)
            S4 = jnp.where(eq, BIGF, S4)
            half_cands.append(m)

    k = len(half_cands) // 2
    U = jnp.concatenate(half_cands[:k], axis=0)
    W = jnp.concatenate(half_cands[k:], axis=0)
    A = jnp.minimum(U, W)
    S2 = jnp.maximum(U, W)
    negs = []
    for _ in range(NUM_NEG):
        m = jnp.min(A, axis=0, keepdims=True)
        eq = A == m
        A = jnp.where(eq, S2, A)
        S2 = jnp.where(eq, BIGF, S2)
        negs.append(m)

    samp = jnp.zeros((P, d2r_ref.shape[1]), jnp.float32)
    for q in range(nq):
        cid, _, _ = _chunk_ids(q)
        s_q = (w00 * (cid == m00) + w01 * (cid == m00 + 1)
               + w10 * (cid == m00 + 32) + w11 * (cid == m00 + 33))
        samp = samp + jax.lax.dot_general(
            s_q.astype(jnp.float32), d2r_ref[0, :, q * Q:(q + 1) * Q],
            (((0,), (1,)), ((), ())),
            preferred_element_type=jnp.float32)

    ns = jnp.sqrt(jnp.sum(samp * samp, axis=1, keepdims=True))
    nsamp = samp / (ns + 1e-8)
    posc = 2.0 - 2.0 * jnp.sum(n1pc * nsamp, axis=1, keepdims=True)
    pos = jnp.transpose(posc, (1, 0))

    acc = jnp.zeros((1, P), jnp.float32)
    for m in negs:
        acc = acc + jnp.maximum(pos - m + MARGIN, 0.0)

    step = pl.program_id(0) * pl.num_programs(1) + pl.program_id(1)

    @pl.when(step == 0)
    def _init_out():
        out_ref[:, :] = jnp.zeros((1, 1), jnp.float32)

    part = jnp.sum(acc, axis=1, keepdims=True)
    upd = out_ref[:, :] + part

    @pl.when(step == nsteps - 1)
    def _scale_out():
        out_ref[:, :] = upd / (nsteps * P * NUM_NEG)

    @pl.when(step < nsteps - 1)
    def _acc_out():
        out_ref[:, :] = upd


@jax.jit
def kernel(kp1, w_kp1, kp1_desc, desc2, homo12):
    b, n, c = kp1_desc.shape
    _, _, hh, ww = desc2.shape
    m = hh * ww
    nq = m // Q
    nb = n // P

    kpt = jnp.transpose(kp1, (0, 2, 1))
    wkpt = jnp.transpose(w_kp1, (0, 2, 1))
    d2r = desc2.reshape(b, c, m)

    grid = (b, nb)
    out = pl.pallas_call(
        functools.partial(_loss_kernel, nq=nq, nsteps=b * nb),
        grid=grid,
        in_specs=[
            pl.BlockSpec((1, 2, P), lambda bi, ri: (bi, 0, ri)),
            pl.BlockSpec((1, 2, P), lambda bi, ri: (bi, 0, ri)),
            pl.BlockSpec((1, P, c), lambda bi, ri: (bi, ri, 0)),
            pl.BlockSpec((1, c, m), lambda bi, ri: (bi, 0, 0)),
            pl.BlockSpec((1, 3, 3), lambda bi, ri: (bi, 0, 0)),
        ],
        out_specs=pl.BlockSpec((1, 1), lambda bi, ri: (0, 0)),
        out_shape=jax.ShapeDtypeStruct((1, 1), jnp.float32),
        scratch_shapes=[pltpu.VMEM((c, m), jnp.float32)],
    )(kpt, wkpt, kp1_desc, d2r, homo12)

    return out.reshape(())


# speedup vs baseline: 1.3633x; 1.0615x over previous
import functools

import jax
import jax.numpy as jnp
from jax.experimental import pallas as pl
from jax.experimental.pallas import tpu as pltpu

GRID = 16.0
MARGIN = 1.0
NUM_NEG = 16
P = 256
Q = 512
BIGF = 3.0e38
BIGI = 2**30


def _chunk_ids(q):
    cid = jax.lax.broadcasted_iota(jnp.int32, (Q, 1), 0) + q * Q
    cx = (cid % 32).astype(jnp.float32) * GRID + GRID / 2.0
    cy = (cid // 32).astype(jnp.float32) * GRID + GRID / 2.0
    return cid, cx, cy


def _top4_axis(p):
    t = (p - GRID / 2.0) / GRID
    k = jnp.floor(t)
    shift = jnp.where(t == k, 1.0, 0.0)
    s = jnp.clip(k - 1.0 - shift, 0.0, 28.0).astype(jnp.int32)
    return [s, s + 1, s + 2, s + 3]


def _nearest4(x, y, nq):
    del nq
    cols = _top4_axis(x)
    rows = _top4_axis(y)
    pm = x * x + y * y
    cand_v, cand_i = [], []
    for ri in rows:
        cyf = ri.astype(jnp.float32) * GRID + GRID / 2.0
        for cj in cols:
            cxf = cj.astype(jnp.float32) * GRID + GRID / 2.0
            cm = cxf * cxf + cyf * cyf
            d2 = (pm + cm) - 2.0 * (cxf * x + cyf * y)
            cand_v.append(jnp.sqrt(jnp.maximum(d2, 1e-12)))
            cand_i.append(ri * 32 + cj)
    V = jnp.concatenate(cand_v, axis=0)
    I = jnp.concatenate(cand_i, axis=0)
    out = []
    for _ in range(4):
        m = jnp.min(V, axis=0, keepdims=True)
        idx = jnp.min(jnp.where(V == m, I, BIGI), axis=0, keepdims=True)
        V = jnp.where(I == idx, BIGF, V)
        out.append(idx)
    return out


def _loss_kernel(kp_ref, wkp_ref, desc_ref, d2r_ref, homo_ref,
                 out_ref, n2_scr, *, nq, nsteps):
    @pl.when(pl.program_id(1) == 0)
    def _build_n2():
        for q in range(nq):
            d = d2r_ref[0, :, q * Q:(q + 1) * Q]
            rn = jnp.sqrt(jnp.sum(d * d, axis=0, keepdims=True))
            n2_scr[:, q * Q:(q + 1) * Q] = d / (rn + 1e-8)

    x = kp_ref[0, 0:1, :]
    y = kp_ref[0, 1:2, :]

    ids1 = _nearest4(x, y, nq)
    h = homo_ref[0]
    ids16 = []
    for idj in ids1:
        cx = (idj % 32).astype(jnp.float32) * GRID + GRID / 2.0
        cy = (idj // 32).astype(jnp.float32) * GRID + GRID / 2.0
        wz = h[2:3, 0:1] * cx + h[2:3, 1:2] * cy + h[2:3, 2:3] + 1e-8
        wx = (h[0:1, 0:1] * cx + h[0:1, 1:2] * cy + h[0:1, 2:3]) / wz
        wy = (h[1:2, 0:1] * cx + h[1:2, 1:2] * cy + h[1:2, 2:3]) / wz
        ids16.extend(_nearest4(wx, wy, nq))

    dpc = desc_ref[0]
    n1pc = dpc / (jnp.sqrt(jnp.sum(dpc * dpc, axis=1, keepdims=True)) + 1e-8)

    sx = jnp.clip(wkp_ref[0, 0:1, :] / GRID - 0.5, 0.0, 31.0)
    sy = jnp.clip(wkp_ref[0, 1:2, :] / GRID - 0.5, 0.0, 31.0)
    x0 = jnp.clip(jnp.floor(sx), 0.0, 30.0)
    y0 = jnp.clip(jnp.floor(sy), 0.0, 30.0)
    fx = sx - x0
    fy = sy - y0
    m00 = y0.astype(jnp.int32) * 32 + x0.astype(jnp.int32)
    w00 = (1.0 - fx) * (1.0 - fy)
    w01 = fx * (1.0 - fy)
    w10 = (1.0 - fx) * fy
    w11 = fx * fy

    half_cands = []
    nh = max(1, nq // 2)
    for hh in range(0, nq, nh):
        l0, l1, l2, l3 = [], [], [], []
        for q in range(hh, hh + nh):
            cid, _, _ = _chunk_ids(q)
            n2q = n2_scr[:, q * Q:(q + 1) * Q]
            g = jax.lax.dot_general(n2q, n1pc, (((0,), (1,)), ((), ())),
                                    preferred_element_type=jnp.float32)
            sim = 2.0 - 2.0 * g
            eqs = [cid == idj for idj in ids16]
            while len(eqs) > 1:
                eqs = [a | b for a, b in zip(eqs[::2], eqs[1::2])]
            sim = jnp.where(eqs[0], BIGF, sim)
            s0, s1 = sim[:Q // 4], sim[Q // 4:Q // 2]
            s2, s3 = sim[Q // 2:3 * Q // 4], sim[3 * Q // 4:]
            a0, a1 = jnp.minimum(s0, s1), jnp.maximum(s0, s1)
            b0, b1 = jnp.minimum(s2, s3), jnp.maximum(s2, s3)
            c0, c2 = jnp.minimum(a0, b0), jnp.maximum(a0, b0)
            c1, c3 = jnp.minimum(a1, b1), jnp.maximum(a1, b1)
            d1, d2 = jnp.minimum(c2, c1), jnp.maximum(c2, c1)
            l0.append(c0)
            l1.append(d1)
            l2.append(d2)
            l3.append(c3)

        A = jnp.concatenate(l0, axis=0)
        S2 = jnp.concatenate(l1, axis=0)
        S3 = jnp.concatenate(l2, axis=0)
        S4 = jnp.concatenate(l3, axis=0)
        for _ in range(NUM_NEG):
            m = jnp.min(A, axis=0, keepdims=True)
            eq = A == m
            A = jnp.where(eq, S2, A)
            S2 = jnp.where(eq, S3, S2)
            S3 = jnp.where(eq, S4, S3)
            S4 = jnp.where(eq, BIGF, S4)
            half_cands.append(m)

    k = len(half_cands) // 2
    U = jnp.concatenate(half_cands[:k], axis=0)
    W = jnp.concatenate(half_cands[k:], axis=0)
    A = jnp.minimum(U, W)
    S2 = jnp.maximum(U, W)
    negs = []
    for _ in range(NUM_NEG):
        m = jnp.min(A, axis=0, keepdims=True)
        eq = A == m
        A = jnp.where(eq, S2, A)
        S2 = jnp.where(eq, BIGF, S2)
        negs.append(m)

    samp = jnp.zeros((P, d2r_ref.shape[1]), jnp.float32)
    for q in range(nq):
        cid, _, _ = _chunk_ids(q)
        s_q = (w00 * (cid == m00) + w01 * (cid == m00 + 1)
               + w10 * (cid == m00 + 32) + w11 * (cid == m00 + 33))
        samp = samp + jax.lax.dot_general(
            s_q.astype(jnp.float32), d2r_ref[0, :, q * Q:(q + 1) * Q],
            (((0,), (1,)), ((), ())),
            preferred_element_type=jnp.float32)

    ns = jnp.sqrt(jnp.sum(samp * samp, axis=1, keepdims=True))
    nsamp = samp / (ns + 1e-8)
    posc = 2.0 - 2.0 * jnp.sum(n1pc * nsamp, axis=1, keepdims=True)
    pos = jnp.transpose(posc, (1, 0))

    acc = jnp.zeros((1, P), jnp.float32)
    for m in negs:
        acc = acc + jnp.maximum(pos - m + MARGIN, 0.0)

    step = pl.program_id(0) * pl.num_programs(1) + pl.program_id(1)

    @pl.when(step == 0)
    def _init_out():
        out_ref[:, :] = jnp.zeros((1, 1), jnp.float32)

    part = jnp.sum(acc, axis=1, keepdims=True)
    upd = out_ref[:, :] + part

    @pl.when(step == nsteps - 1)
    def _scale_out():
        out_ref[:, :] = upd / (nsteps * P * NUM_NEG)

    @pl.when(step < nsteps - 1)
    def _acc_out():
        out_ref[:, :] = upd


@jax.jit
def kernel(kp1, w_kp1, kp1_desc, desc2, homo12):
    b, n, c = kp1_desc.shape
    _, _, hh, ww = desc2.shape
    m = hh * ww
    nq = m // Q
    nb = n // P

    kpt = jnp.transpose(kp1, (0, 2, 1))
    wkpt = jnp.transpose(w_kp1, (0, 2, 1))
    d2r = desc2.reshape(b, c, m)

    grid = (b, nb)
    out = pl.pallas_call(
        functools.partial(_loss_kernel, nq=nq, nsteps=b * nb),
        grid=grid,
        in_specs=[
            pl.BlockSpec((1, 2, P), lambda bi, ri: (bi, 0, ri)),
            pl.BlockSpec((1, 2, P), lambda bi, ri: (bi, 0, ri)),
            pl.BlockSpec((1, P, c), lambda bi, ri: (bi, ri, 0)),
            pl.BlockSpec((1, c, m), lambda bi, ri: (bi, 0, 0)),
            pl.BlockSpec((1, 3, 3), lambda bi, ri: (bi, 0, 0)),
        ],
        out_specs=pl.BlockSpec((1, 1), lambda bi, ri: (0, 0)),
        out_shape=jax.ShapeDtypeStruct((1, 1), jnp.float32),
        scratch_shapes=[pltpu.VMEM((c, m), jnp.float32)],
    )(kpt, wkpt, kp1_desc, d2r, homo12)

    return out.reshape(())
